# baseline (speedup 1.0000x reference)
import functools

import jax
import jax.numpy as jnp
from jax import lax
from jax.experimental import pallas as pl
from jax.experimental.pallas import tpu as pltpu
from jax.experimental.pallas import tpu_sc as plsc

_L = 16
_PW = 128
_WIN = 4
_CAND = 1024
_NS = 8


def _mesh_kernel(out_type, scratch):
    mesh = plsc.VectorSubcoreMesh(core_axis_name="c", subcore_axis_name="s")
    return functools.partial(
        pl.kernel,
        mesh=mesh,
        compiler_params=pltpu.CompilerParams(needs_layout_passes=False),
        out_type=out_type,
        scratch_types=scratch,
    )


def _gather_vectors(user_ids, item_ids, ut, it, utail, itail):
    B = user_ids.shape[0]
    D = ut.shape[0]
    nu, ni = ut.shape[1], it.shape[1]
    n_up = nu // _PW
    n_ip = ni // _PW
    u_tstart = n_up * _PW
    i_tstart = n_ip * _PW
    u_nt, i_nt = nu - u_tstart, ni - i_tstart
    assert utail.shape == (u_nt, D) and itail.shape == (i_nt, D)

    info = plsc.get_sparse_core_info()
    nw = info.num_cores * info.num_subcores
    b_per_w = B // nw
    u_span = -(-n_up // nw)
    i_span = -(-n_ip // nw)
    u_nwin = -(-u_span // _WIN)
    i_nwin = -(-i_span // _WIN)
    NSTAGE = B + 2 * _L
    DUMP = B + _L

    staged = jax.ShapeDtypeStruct((NSTAGE, _PW), jnp.float32)

    @_mesh_kernel(
        (staged, staged),
        [
            pltpu.VMEM((B,), jnp.int32),
            pltpu.VMEM((B,), jnp.int32),
            pltpu.VMEM((_CAND,), jnp.int32),
            pltpu.VMEM((_CAND,), jnp.int32),
            pltpu.VMEM((2, _WIN * D, _PW), jnp.float32),
            pltpu.VMEM((_NS, _L, _PW), jnp.float32),
            pltpu.VMEM((u_nt, D), jnp.float32),
            pltpu.VMEM((i_nt, D), jnp.float32),
            pltpu.SemaphoreType.DMA((2,)),
            pltpu.SemaphoreType.DMA((_NS,)),
        ],
    )
    def k(uids_hbm, iids_hbm, ut_hbm, it_hbm, utail_hbm, itail_hbm,
          uvec_hbm, ivec_hbm, uid_v, iid_v, cid, cpos, wbuf, stage,
          utb, itb, wsem, ssem):
        wid = lax.axis_index("s") * info.num_cores + lax.axis_index("c")
        pltpu.sync_copy(uids_hbm, uid_v)
        pltpu.sync_copy(iids_hbm, iid_v)
        pltpu.sync_copy(utail_hbm, utb)
        pltpu.sync_copy(itail_hbm, itb)

        lane = lax.iota(jnp.int32, _L)
        lane128 = lane * _PW

        def build_cands(ids_v, p_lo, p_hi, tstart):
            def body(q, cnt):
                v = ids_v[pl.ds(q * _L, _L)]
                pan = v >> 7
                m = (pan >= p_lo) & (pan < p_hi) & (v < tstart)
                off = jnp.minimum(cnt, _CAND - _L)
                plsc.store_compressed(cid.at[pl.ds(off, _L)], v, mask=m)
                plsc.store_compressed(
                    cpos.at[pl.ds(off, _L)], q * _L + lane, mask=m)
                return cnt + plsc.all_reduce_population_count(m)[0]
            return lax.fori_loop(0, B // _L, body, 0)

        def scatter_stage(sbuf, svec, out_hbm, pos, sem):
            return pltpu.async_copy(out_hbm.at[pos], sbuf, sem)

        def stream_phase(tab_hbm, out_hbm, n_pan, p_lo, n_win, cnt):
            def win_panels(w, buf):
                cps = []
                for j in range(_WIN):
                    p = jnp.minimum(p_lo + w * _WIN + j, n_pan - 1)
                    off = pl.multiple_of(p * _PW, _PW)
                    cps.append(pltpu.async_copy(
                        tab_hbm.at[:, pl.ds(off, _PW)],
                        wbuf.at[buf, pl.ds(j * D, D)], wsem.at[buf]))
                return cps

            def win_wait(buf):
                for j in range(_WIN):
                    pltpu.make_async_copy(
                        tab_hbm.at[:, pl.ds(0, _PW)],
                        wbuf.at[buf, pl.ds(j * D, D)], wsem.at[buf]).wait()

            win_panels(0, 0)

            def win_body(w, carry):
                buf = w % 2
                nxt = (w + 1) % 2

                @pl.when(w + 1 < n_win)
                def _():
                    win_panels(w + 1, nxt)

                win_wait(buf)
                wp0 = p_lo + w * _WIN

                def scan_body(q, sb):
                    v = cid[pl.ds(q * _L, _L)]
                    pos_q = cpos[pl.ds(q * _L, _L)]
                    pan = v >> 7
                    valid = (q * _L + lane) < cnt
                    m = (pan >= wp0) & (pan < wp0 + _WIN) & valid

                    def extract():
                        inp = jnp.clip(pan - wp0, 0, _WIN - 1)
                        col = v & (_PW - 1)
                        rowbase = inp * D
                        s = sb % _NS
                        pltpu.make_async_copy(
                            stage.at[s], out_hbm.at[pl.ds(0, _L)],
                            ssem.at[s]).wait()
                        for d in range(D):
                            val = plsc.load_gather(
                                wbuf.at[buf], [rowbase + d, col])
                            plsc.store_scatter(
                                stage.at[s], [lane, jnp.full(
                                    (_L,), d, jnp.int32)], val)
                        pos = jnp.where(m, pos_q, DUMP)
                        pltpu.async_copy(stage.at[s],
                                         out_hbm.at[pos], ssem.at[s])

                    pl.when(jnp.any(m))(extract)
                    return sb + 1

                nvec = (cnt + _L - 1) >> 4
                return lax.fori_loop(0, nvec, scan_body, carry)

            return lax.fori_loop(0, n_win, win_body, 0)

        def tail_phase(ids_v, out_hbm, tb, tstart, nt, sb0):
            base = wid * b_per_w

            def body(q, sb):
                v = ids_v[pl.ds(base + q * _L, _L)]
                m = v >= tstart

                def extract():
                    s = sb % _NS
                    row = jnp.clip(v - tstart, 0, nt - 1)
                    pltpu.make_async_copy(
                        stage.at[s], out_hbm.at[pl.ds(0, _L)],
                        ssem.at[s]).wait()
                    for d in range(D):
                        dv = jnp.full((_L,), d, jnp.int32)
                        val = plsc.load_gather(tb, [row, dv])
                        plsc.store_scatter(stage.at[s], [lane, dv], val)
                    pos = jnp.where(m, base + q * _L + lane, DUMP)
                    pltpu.async_copy(stage.at[s], out_hbm.at[pos],
                                     ssem.at[s])

                pl.when(jnp.any(m))(extract)
                return sb + 1

            return lax.fori_loop(0, b_per_w // _L, body, sb0)

        for s in range(_NS):
            pltpu.async_copy(stage.at[s], uvec_hbm.at[pl.ds(B, _L)],
                             ssem.at[s])

        u_lo = wid * u_span
        cnt_u = build_cands(uid_v, u_lo, u_lo + u_span, u_tstart)
        sb = stream_phase(ut_hbm, uvec_hbm, n_up, u_lo, u_nwin, cnt_u)
        sb = tail_phase(uid_v, uvec_hbm, utb, u_tstart, u_nt, sb)

        i_lo = wid * i_span
        cnt_i = build_cands(iid_v, i_lo, i_lo + i_span, i_tstart)
        sb = stream_phase(it_hbm, ivec_hbm, n_ip, i_lo, i_nwin, cnt_i)
        sb = tail_phase(iid_v, ivec_hbm, itb, i_tstart, i_nt, sb)

        for s in range(_NS):
            pltpu.make_async_copy(stage.at[s], uvec_hbm.at[pl.ds(0, _L)],
                                  ssem.at[s]).wait()

    return k(user_ids, item_ids, ut, it, utail, itail)


def _dot_kernel(uvec, ivec, B):
    D = 64
    info = plsc.get_sparse_core_info()
    nw = info.num_cores * info.num_subcores
    b_per_w = B // nw
    CH = 64

    @_mesh_kernel(
        jax.ShapeDtypeStruct((B,), jnp.float32),
        [
            pltpu.VMEM((CH, _PW), jnp.float32),
            pltpu.VMEM((CH, _PW), jnp.float32),
            pltpu.VMEM((b_per_w,), jnp.float32),
            pltpu.SemaphoreType.DMA,
        ],
    )
    def k(uvec_hbm, ivec_hbm, out_hbm, ub, ib, out_v, sem):
        wid = lax.axis_index("s") * info.num_cores + lax.axis_index("c")
        base = wid * b_per_w
        lane = lax.iota(jnp.int32, _L)

        def chunk(c, carry):
            r0 = base + c * CH
            cu = pltpu.async_copy(uvec_hbm.at[pl.ds(r0, CH), :], ub, sem)
            ci = pltpu.async_copy(ivec_hbm.at[pl.ds(r0, CH), :], ib, sem)
            cu.wait()
            ci.wait()

            def blk(b2, carry2):
                rows = b2 * _L + lane
                acc = jnp.zeros((_L,), jnp.float32)
                for d in range(D):
                    dv = jnp.full((_L,), d, jnp.int32)
                    u = plsc.load_gather(ub, [rows, dv])
                    v = plsc.load_gather(ib, [rows, dv])
                    acc = acc + u * v
                out_v[pl.ds(c * CH + b2 * _L, _L)] = acc
                return carry2

            lax.fori_loop(0, CH // _L, blk, 0)
            return carry

        lax.fori_loop(0, b_per_w // CH, chunk, 0)
        pltpu.sync_copy(out_v, out_hbm.at[pl.ds(base, b_per_w)])

    return k(uvec, ivec)


def kernel(user_ids, item_ids, user_table, item_table):
    B = user_ids.shape[0]
    nu = user_table.shape[0]
    ni = item_table.shape[0]
    u_tstart = (nu // _PW) * _PW
    i_tstart = (ni // _PW) * _PW
    uvec, ivec = _gather_vectors(
        user_ids.astype(jnp.int32), item_ids.astype(jnp.int32),
        user_table.T, item_table.T,
        user_table[u_tstart:], item_table[i_tstart:])
    scores = _dot_kernel(uvec, ivec, B)
    return scores.reshape(B, 1)


# speedup vs baseline: 1.0003x; 1.0003x over previous
import functools

import jax
import jax.numpy as jnp
from jax import lax
from jax.experimental import pallas as pl
from jax.experimental.pallas import tpu as pltpu
from jax.experimental.pallas import tpu_sc as plsc

_L = 16
_BW = 512
_CAND = 1024
_NS = 8
_D = 64


def _mesh_kernel(out_type, scratch):
    mesh = plsc.VectorSubcoreMesh(core_axis_name="c", subcore_axis_name="s")
    return functools.partial(
        pl.kernel,
        mesh=mesh,
        compiler_params=pltpu.CompilerParams(needs_layout_passes=False),
        out_type=out_type,
        scratch_types=scratch,
    )


def _gather_vectors(user_ids, item_ids, ut, it):
    B = user_ids.shape[0]
    D = ut.shape[0]
    nu, ni = ut.shape[1], it.shape[1]
    n_ub = nu // _BW
    n_ib = ni // _BW
    u_tstart = n_ub * _BW
    i_tstart = n_ib * _BW

    info = plsc.get_sparse_core_info()
    nw = info.num_cores * info.num_subcores
    u_span = -(-n_ub // nw)
    i_span = -(-n_ib // nw)
    NSTAGE = B + 2 * _L
    DUMP = B + _L

    staged = jax.ShapeDtypeStruct((NSTAGE, 2 * D), jnp.float32)

    @_mesh_kernel(
        (staged, staged),
        [
            pltpu.VMEM((B,), jnp.int32),
            pltpu.VMEM((B,), jnp.int32),
            pltpu.VMEM((_CAND,), jnp.int32),
            pltpu.VMEM((_CAND,), jnp.int32),
            pltpu.VMEM((2, D, _BW), jnp.float32),
            pltpu.VMEM((_NS, _L, 2 * D), jnp.float32),
            pltpu.SemaphoreType.DMA((2,)),
            pltpu.SemaphoreType.DMA((_NS,)),
        ],
    )
    def k(uids_hbm, iids_hbm, ut_hbm, it_hbm, uvec_hbm, ivec_hbm,
          uid_v, iid_v, cid, cpos, wbuf, stage, wsem, ssem):
        wid = lax.axis_index("s") * info.num_cores + lax.axis_index("c")
        pltpu.sync_copy(uids_hbm, uid_v)
        pltpu.sync_copy(iids_hbm, iid_v)

        lane = lax.iota(jnp.int32, _L)

        for s in range(_NS):
            pltpu.async_copy(stage.at[s], uvec_hbm.at[pl.ds(B, _L)],
                             ssem.at[s])

        def build_cands(ids_v, b_lo, b_hi, tstart):
            def body(q, cnt):
                v = ids_v[pl.ds(q * _L, _L)]
                blk = v >> 9
                m = (blk >= b_lo) & (blk < b_hi) & (v < tstart)
                off = jnp.minimum(cnt, _CAND - _L)
                plsc.store_compressed(cid.at[pl.ds(off, _L)], v, mask=m)
                plsc.store_compressed(
                    cpos.at[pl.ds(off, _L)], q * _L + lane, mask=m)
                return cnt + plsc.all_reduce_population_count(m)[0]
            return lax.fori_loop(0, B // _L, body, 0)

        def stream_phase(tab_hbm, out_hbm, n_blk, b_lo, span, cnt, sb0):
            def start_blk(b, buf):
                blk = jnp.minimum(b_lo + b, n_blk - 1)
                off = pl.multiple_of(blk * _BW, _BW)
                for j in range(D // 8):
                    pltpu.async_copy(
                        tab_hbm.at[pl.ds(j * 8, 8), pl.ds(off, _BW)],
                        wbuf.at[buf, pl.ds(j * 8, 8), :], wsem.at[buf])

            def wait_blk(buf):
                for j in range(D // 8):
                    pltpu.make_async_copy(
                        tab_hbm.at[pl.ds(0, 8), pl.ds(0, _BW)],
                        wbuf.at[buf, pl.ds(j * 8, 8), :],
                        wsem.at[buf]).wait()

            start_blk(0, 0)
            nvec = (cnt + _L - 1) >> 4

            def blk_body(b, sb):
                buf = b % 2

                @pl.when(b + 1 < span)
                def _():
                    start_blk(b + 1, (b + 1) % 2)

                wait_blk(buf)
                cur = b_lo + b

                def scan_body(q, sb2):
                    v = cid[pl.ds(q * _L, _L)]
                    pos_q = cpos[pl.ds(q * _L, _L)]
                    valid = (q * _L + lane) < cnt
                    m = ((v >> 9) == cur) & valid

                    def extract():
                        s = sb2 % _NS
                        col = v & (_BW - 1)
                        pltpu.make_async_copy(
                            stage.at[s], out_hbm.at[pl.ds(0, _L)],
                            ssem.at[s]).wait()
                        for d in range(D):
                            val = plsc.load_gather(
                                wbuf.at[buf], [jnp.full((_L,), d, jnp.int32),
                                               col])
                            plsc.store_scatter(
                                stage.at[s],
                                [lane, jnp.full((_L,), d, jnp.int32)], val)
                        pos = jnp.where(m, pos_q, DUMP)
                        pltpu.async_copy(stage.at[s], out_hbm.at[pos],
                                         ssem.at[s])

                    pl.when(jnp.any(m))(extract)
                    return sb2 + 1

                return lax.fori_loop(0, nvec, scan_body, sb)

            return lax.fori_loop(0, span, blk_body, sb0)

        u_lo = wid * u_span
        cnt_u = build_cands(uid_v, u_lo, u_lo + u_span, u_tstart)
        sb = stream_phase(ut_hbm, uvec_hbm, n_ub, u_lo, u_span, cnt_u, 0)

        i_lo = wid * i_span
        cnt_i = build_cands(iid_v, i_lo, i_lo + i_span, i_tstart)
        stream_phase(it_hbm, ivec_hbm, n_ib, i_lo, i_span, cnt_i, sb)

        for s in range(_NS):
            pltpu.make_async_copy(stage.at[s], uvec_hbm.at[pl.ds(0, _L)],
                                  ssem.at[s]).wait()

    return k(user_ids, item_ids, ut, it)


def _dot_kernel(user_ids, item_ids, uvec, ivec, utail, itail, B,
                u_tstart, i_tstart):
    D = _D
    u_nt = utail.shape[0]
    i_nt = itail.shape[0]
    info = plsc.get_sparse_core_info()
    nw = info.num_cores * info.num_subcores
    b_per_w = B // nw
    CH = 64

    @_mesh_kernel(
        jax.ShapeDtypeStruct((B,), jnp.float32),
        [
            pltpu.VMEM((b_per_w,), jnp.int32),
            pltpu.VMEM((b_per_w,), jnp.int32),
            pltpu.VMEM((CH, 2 * D), jnp.float32),
            pltpu.VMEM((CH, 2 * D), jnp.float32),
            pltpu.VMEM((u_nt, D), jnp.float32),
            pltpu.VMEM((i_nt, D), jnp.float32),
            pltpu.VMEM((b_per_w,), jnp.float32),
            pltpu.SemaphoreType.DMA,
        ],
    )
    def k(uids_hbm, iids_hbm, uvec_hbm, ivec_hbm, utail_hbm, itail_hbm,
          out_hbm, uid_v, iid_v, ub, ib, utb, itb, out_v, sem):
        wid = lax.axis_index("s") * info.num_cores + lax.axis_index("c")
        base = wid * b_per_w
        pltpu.sync_copy(uids_hbm.at[pl.ds(base, b_per_w)], uid_v)
        pltpu.sync_copy(iids_hbm.at[pl.ds(base, b_per_w)], iid_v)
        pltpu.sync_copy(utail_hbm, utb)
        pltpu.sync_copy(itail_hbm, itb)
        lane = lax.iota(jnp.int32, _L)

        def chunk(c, carry):
            r0 = base + c * CH
            cu = pltpu.async_copy(uvec_hbm.at[pl.ds(r0, CH), :], ub, sem)
            ci = pltpu.async_copy(ivec_hbm.at[pl.ds(r0, CH), :], ib, sem)
            cu.wait()
            ci.wait()

            def blk(b2, carry2):
                rows = b2 * _L + lane
                o0 = c * CH + b2 * _L
                uids = uid_v[pl.ds(o0, _L)]
                iids = iid_v[pl.ds(o0, _L)]
                um = uids >= u_tstart
                im = iids >= i_tstart
                any_tail = jnp.any(um | im)

                def dot_plain():
                    acc = jnp.zeros((_L,), jnp.float32)
                    for d in range(D):
                        dv = jnp.full((_L,), d, jnp.int32)
                        u = plsc.load_gather(ub, [rows, dv])
                        v = plsc.load_gather(ib, [rows, dv])
                        acc = acc + u * v
                    out_v[pl.ds(o0, _L)] = acc

                def dot_tail():
                    urow = jnp.clip(uids - u_tstart, 0, u_nt - 1)
                    irow = jnp.clip(iids - i_tstart, 0, i_nt - 1)
                    acc = jnp.zeros((_L,), jnp.float32)
                    for d in range(D):
                        dv = jnp.full((_L,), d, jnp.int32)
                        u = plsc.load_gather(ub, [rows, dv])
                        v = plsc.load_gather(ib, [rows, dv])
                        tu = plsc.load_gather(utb, [urow, dv])
                        tv = plsc.load_gather(itb, [irow, dv])
                        u = jnp.where(um, tu, u)
                        v = jnp.where(im, tv, v)
                        acc = acc + u * v
                    out_v[pl.ds(o0, _L)] = acc

                pl.when(any_tail)(dot_tail)
                pl.when(jnp.logical_not(any_tail))(dot_plain)
                return carry2

            lax.fori_loop(0, CH // _L, blk, 0)
            return carry

        lax.fori_loop(0, b_per_w // CH, chunk, 0)
        pltpu.sync_copy(out_v, out_hbm.at[pl.ds(base, b_per_w)])

    return k(user_ids, item_ids, uvec, ivec, utail, itail)


def kernel(user_ids, item_ids, user_table, item_table):
    B = user_ids.shape[0]
    nu = user_table.shape[0]
    ni = item_table.shape[0]
    u_tstart = (nu // _BW) * _BW
    i_tstart = (ni // _BW) * _BW
    uids = user_ids.astype(jnp.int32)
    iids = item_ids.astype(jnp.int32)
    uvec, ivec = _gather_vectors(uids, iids, user_table.T, item_table.T)
    scores = _dot_kernel(uids, iids, uvec, ivec,
                         user_table[u_tstart:], item_table[i_tstart:], B,
                         u_tstart, i_tstart)
    return scores.reshape(B, 1)


# speedup vs baseline: 17.3115x; 17.3061x over previous
import functools

import jax
import jax.numpy as jnp
from jax import lax
from jax.experimental import pallas as pl
from jax.experimental.pallas import tpu as pltpu
from jax.experimental.pallas import tpu_sc as plsc

_L = 16


def _scores_sc(user_ids, item_ids, user_table, item_table):
    B = user_ids.shape[0]
    D = user_table.shape[1]
    info = plsc.get_sparse_core_info()
    nw = info.num_cores * info.num_subcores
    b_per_w = B // nw

    mesh = plsc.VectorSubcoreMesh(core_axis_name="c", subcore_axis_name="s")

    @functools.partial(
        pl.kernel,
        mesh=mesh,
        compiler_params=pltpu.CompilerParams(
            needs_layout_passes=False, use_tc_tiling_on_sc=False),
        out_type=jax.ShapeDtypeStruct((B,), jnp.float32),
        scratch_types=[
            pltpu.VMEM((b_per_w,), jnp.int32),
            pltpu.VMEM((b_per_w,), jnp.int32),
            pltpu.VMEM((b_per_w, D), jnp.float32),
            pltpu.VMEM((b_per_w, D), jnp.float32),
            pltpu.VMEM((_L * _L,), jnp.float32),
            pltpu.VMEM((b_per_w,), jnp.float32),
            pltpu.SemaphoreType.DMA,
            pltpu.SemaphoreType.DMA,
        ],
    )
    def k(uids_hbm, iids_hbm, utab_hbm, itab_hbm, out_hbm,
          uidx_v, iidx_v, urows_v, irows_v, tv, out_v, sem_u, sem_i):
        wid = lax.axis_index("s") * info.num_cores + lax.axis_index("c")
        base = wid * b_per_w
        pltpu.sync_copy(uids_hbm.at[pl.ds(base, b_per_w)], uidx_v)
        pltpu.sync_copy(iids_hbm.at[pl.ds(base, b_per_w)], iidx_v)
        cu = pltpu.async_copy(utab_hbm.at[uidx_v], urows_v, sem_u)
        ci = pltpu.async_copy(itab_hbm.at[iidx_v], irows_v, sem_i)
        cu.wait()
        ci.wait()

        lane = lax.iota(jnp.int32, _L)

        def blk_body(blk, carry):
            row0 = blk * _L
            for rj in range(_L):
                r = row0 + rj
                p = urows_v[r, pl.ds(0, _L)] * irows_v[r, pl.ds(0, _L)]
                for c in range(1, D // _L):
                    p = p + (urows_v[r, pl.ds(c * _L, _L)]
                             * irows_v[r, pl.ds(c * _L, _L)])
                plsc.store_scatter(tv, [lane * _L + rj], p)
            acc = tv[pl.ds(0, _L)]
            for l in range(1, _L):
                acc = acc + tv[pl.ds(l * _L, _L)]
            out_v[pl.ds(row0, _L)] = acc
            return carry

        lax.fori_loop(0, b_per_w // _L, blk_body, 0)
        pltpu.sync_copy(out_v, out_hbm.at[pl.ds(base, b_per_w)])

    return k(user_ids, item_ids, user_table, item_table)


def kernel(user_ids, item_ids, user_table, item_table):
    B = user_ids.shape[0]
    scores = _scores_sc(user_ids.astype(jnp.int32), item_ids.astype(jnp.int32),
                        user_table, item_table)
    return scores.reshape(B, 1)
